# baseline (speedup 1.0000x reference)
import jax
import jax.numpy as jnp
from jax.experimental import pallas as pl

_TOP_K = 8


def _gate_kernel(hs_ref, w_ref, idx_ref, wgt_ref):
    x = hs_ref[...]
    w = w_ref[...]
    logits = jax.lax.dot_general(
        x, w, (((1,), (1,)), ((), ())), preferred_element_type=jnp.float32
    )
    t, e = logits.shape
    iota_f = jax.lax.broadcasted_iota(jnp.int32, (t, e), 1).astype(jnp.float32)
    cur = logits
    vals = []
    idxs = []
    for _ in range(_TOP_K):
        m = jnp.max(cur, axis=1, keepdims=True)
        is_max = cur == m
        i = jnp.min(jnp.where(is_max, iota_f, float(e)), axis=1, keepdims=True)
        vals.append(m)
        idxs.append(i)
        cur = jnp.where(iota_f == i, -jnp.inf, cur)
    v = jnp.concatenate(vals, axis=1)
    ii = jnp.concatenate(idxs, axis=1).astype(jnp.int32)
    ex = jnp.exp(v - v[:, :1])
    wgt = ex / jnp.sum(ex, axis=1, keepdims=True)
    idx_ref[...] = ii
    wgt_ref[...] = wgt


def kernel(hidden_states, weight):
    b, s, h = hidden_states.shape
    hs = hidden_states.reshape(-1, h)
    n = hs.shape[0]
    e = weight.shape[0]
    t = 256
    grid = n // t
    idx, wgt = pl.pallas_call(
        _gate_kernel,
        grid=(grid,),
        in_specs=[
            pl.BlockSpec((t, h), lambda i: (i, 0)),
            pl.BlockSpec((e, h), lambda i: (0, 0)),
        ],
        out_specs=[
            pl.BlockSpec((t, _TOP_K), lambda i: (i, 0)),
            pl.BlockSpec((t, _TOP_K), lambda i: (i, 0)),
        ],
        out_shape=[
            jax.ShapeDtypeStruct((n, _TOP_K), jnp.int32),
            jax.ShapeDtypeStruct((n, _TOP_K), jnp.float32),
        ],
    )(hs, weight)
    return idx, wgt


# speedup vs baseline: 1.4885x; 1.4885x over previous
import jax
import jax.numpy as jnp
from jax.experimental import pallas as pl

_TOP_K = 8


def _gate_kernel(hs_ref, w_ref, idx_ref, wgt_ref):
    x = hs_ref[...]
    w = w_ref[...]
    logits = jax.lax.dot_general(
        x, w, (((1,), (1,)), ((), ())), preferred_element_type=jnp.float32
    )
    t, e = logits.shape
    iota_f = jax.lax.broadcasted_iota(jnp.int32, (t, e), 1).astype(jnp.float32)
    cur = logits
    vals = []
    idxs = []
    for _ in range(_TOP_K):
        m = jnp.max(cur, axis=1, keepdims=True)
        is_max = cur == m
        i = jnp.min(jnp.where(is_max, iota_f, float(e)), axis=1, keepdims=True)
        vals.append(m)
        idxs.append(i)
        cur = jnp.where(iota_f == i, -jnp.inf, cur)
    v = jnp.concatenate(vals, axis=1)
    ii = jnp.concatenate(idxs, axis=1).astype(jnp.int32)
    ex = jnp.exp(v - v[:, :1])
    wgt = ex / jnp.sum(ex, axis=1, keepdims=True)
    idx_ref[...] = ii
    wgt_ref[...] = wgt


def kernel(hidden_states, weight):
    b, s, h = hidden_states.shape
    hs = hidden_states.reshape(-1, h)
    n = hs.shape[0]
    e = weight.shape[0]
    t = 1024
    grid = n // t
    idx, wgt = pl.pallas_call(
        _gate_kernel,
        grid=(grid,),
        in_specs=[
            pl.BlockSpec((t, h), lambda i: (i, 0)),
            pl.BlockSpec((e, h), lambda i: (0, 0)),
        ],
        out_specs=[
            pl.BlockSpec((t, _TOP_K), lambda i: (i, 0)),
            pl.BlockSpec((t, _TOP_K), lambda i: (i, 0)),
        ],
        out_shape=[
            jax.ShapeDtypeStruct((n, _TOP_K), jnp.int32),
            jax.ShapeDtypeStruct((n, _TOP_K), jnp.float32),
        ],
    )(hs, weight)
    return idx, wgt


# speedup vs baseline: 1.8795x; 1.2627x over previous
import jax
import jax.numpy as jnp
from jax.experimental import pallas as pl

_TOP_K = 8


def _gate_kernel(hs_ref, w_ref, idx_ref, wgt_ref):
    x = hs_ref[...]
    w = w_ref[...]
    logits = jax.lax.dot_general(
        x, w, (((1,), (1,)), ((), ())), preferred_element_type=jnp.float32
    )
    t, e = logits.shape
    iota_f = jax.lax.broadcasted_iota(jnp.int32, (t, e), 1).astype(jnp.float32)
    cur = logits
    vals = []
    idxs = []
    for _ in range(_TOP_K):
        m = jnp.max(cur, axis=1, keepdims=True)
        is_max = cur == m
        i = jnp.min(jnp.where(is_max, iota_f, float(e)), axis=1, keepdims=True)
        vals.append(m)
        idxs.append(i)
        cur = jnp.where(iota_f == i, -jnp.inf, cur)
    v = jnp.concatenate(vals, axis=1)
    ii = jnp.concatenate(idxs, axis=1).astype(jnp.int32)
    ex = jnp.exp(v - v[:, :1])
    wgt = ex / jnp.sum(ex, axis=1, keepdims=True)
    idx_ref[...] = ii
    wgt_ref[...] = wgt


def kernel(hidden_states, weight):
    b, s, h = hidden_states.shape
    hs = hidden_states.reshape(-1, h)
    n = hs.shape[0]
    e = weight.shape[0]
    t = 2048
    grid = n // t
    idx, wgt = pl.pallas_call(
        _gate_kernel,
        grid=(grid,),
        in_specs=[
            pl.BlockSpec((t, h), lambda i: (i, 0)),
            pl.BlockSpec((e, h), lambda i: (0, 0)),
        ],
        out_specs=[
            pl.BlockSpec((t, _TOP_K), lambda i: (i, 0)),
            pl.BlockSpec((t, _TOP_K), lambda i: (i, 0)),
        ],
        out_shape=[
            jax.ShapeDtypeStruct((n, _TOP_K), jnp.int32),
            jax.ShapeDtypeStruct((n, _TOP_K), jnp.float32),
        ],
    )(hs, weight)
    return idx, wgt


# speedup vs baseline: 1.8918x; 1.0066x over previous
import jax
import jax.numpy as jnp
from jax.experimental import pallas as pl

_TOP_K = 8
_T = 2048
_SPLIT = 4


def _gate_kernel(*refs):
    hs_refs = refs[:_SPLIT]
    w_ref = refs[_SPLIT]
    idx_ref, wgt_ref = refs[_SPLIT + 1], refs[_SPLIT + 2]
    w = w_ref[...]
    hc = hs_refs[0].shape[1]
    logits = None
    for j in range(_SPLIT):
        part = jax.lax.dot_general(
            hs_refs[j][...],
            w[:, j * hc : (j + 1) * hc],
            (((1,), (1,)), ((), ())),
            preferred_element_type=jnp.float32,
        )
        logits = part if logits is None else logits + part
    t, e = logits.shape
    iota_f = jax.lax.broadcasted_iota(jnp.int32, (t, e), 1).astype(jnp.float32)
    cur = logits
    vals = []
    idxs = []
    for _ in range(_TOP_K):
        m = jnp.max(cur, axis=1, keepdims=True)
        is_max = cur == m
        i = jnp.min(jnp.where(is_max, iota_f, float(e)), axis=1, keepdims=True)
        vals.append(m)
        idxs.append(i)
        cur = jnp.where(iota_f == i, -jnp.inf, cur)
    v = jnp.concatenate(vals, axis=1)
    ii = jnp.concatenate(idxs, axis=1).astype(jnp.int32)
    ex = jnp.exp(v - v[:, :1])
    wgt = ex / jnp.sum(ex, axis=1, keepdims=True)
    idx_ref[...] = ii
    wgt_ref[...] = wgt


def kernel(hidden_states, weight):
    b, s, h = hidden_states.shape
    hs = hidden_states.reshape(-1, h)
    n = hs.shape[0]
    e = weight.shape[0]
    hc = h // _SPLIT
    grid = n // _T
    in_specs = [
        pl.BlockSpec((_T, hc), lambda i, j=j: (i, j)) for j in range(_SPLIT)
    ] + [pl.BlockSpec((e, h), lambda i: (0, 0))]
    idx, wgt = pl.pallas_call(
        _gate_kernel,
        grid=(grid,),
        in_specs=in_specs,
        out_specs=[
            pl.BlockSpec((_T, _TOP_K), lambda i: (i, 0)),
            pl.BlockSpec((_T, _TOP_K), lambda i: (i, 0)),
        ],
        out_shape=[
            jax.ShapeDtypeStruct((n, _TOP_K), jnp.int32),
            jax.ShapeDtypeStruct((n, _TOP_K), jnp.float32),
        ],
    )(*([hs] * _SPLIT), weight)
    return idx, wgt
